# baseline (speedup 1.0000x reference)
import functools

import jax
import jax.numpy as jnp
from jax import lax
from jax.experimental import pallas as pl
from jax.experimental.pallas import tpu as pltpu
from jax.experimental.pallas import tpu_sc as plsc

_VOCAB = 1000000
_D = 64
_B = 16384
_LANES = 16

_NC = 2
_NS = 16
_NW = _NC * _NS

_CH = 128
_NCHUNK = _B // (_NW * _CH)
_G = _NW * _NCHUNK

_VBLK = 384
_NBLK = _VOCAB // _VBLK
_VTAIL = _VOCAB - _NBLK * _VBLK
_BLK_W = -(-_NBLK // _NW)
_BLK_LO = _NBLK - (_BLK_W - 1) * _NW


def _make_transpose_call():
    mesh = plsc.VectorSubcoreMesh(core_axis_name="c", subcore_axis_name="s")

    @functools.partial(
        pl.kernel,
        out_type=jax.ShapeDtypeStruct((_VOCAB * _D,), jnp.float32),
        mesh=mesh,
        compiler_params=pltpu.CompilerParams(
            use_tc_tiling_on_sc=True, needs_layout_passes=False
        ),
        scratch_types=[
            pltpu.VMEM((2, _D, _VBLK + 1), jnp.float32),
            pltpu.VMEM((2, _VBLK * _D), jnp.float32),
            pltpu.VMEM((_VTAIL * _D,), jnp.float32),
            pltpu.SemaphoreType.DMA,
            pltpu.SemaphoreType.DMA((2,)),
        ],
    )
    def call(tT_hbm, tail_hbm, out_hbm, in_v, tr_v, tail_v, isem, osem):
        wid = lax.axis_index("s") * _NC + lax.axis_index("c")
        nblk = jnp.where(wid < _BLK_LO, _BLK_W, _BLK_W - 1)
        lanes = lax.iota(jnp.int32, _LANES)

        def cstart(n):
            return pl.multiple_of((wid + _NW * n) * _VBLK, 128)

        pltpu.async_copy(
            tT_hbm.at[:, pl.ds(cstart(0), _VBLK)],
            in_v.at[0, :, pl.ds(0, _VBLK)], isem,
        )

        def blk(n, _):
            buf = lax.rem(n, 2)
            pltpu.make_async_copy(
                tT_hbm.at[:, pl.ds(0, _VBLK)],
                in_v.at[buf, :, pl.ds(0, _VBLK)], isem
            ).wait()

            @pl.when(n + 1 < nblk)
            def _():
                pltpu.async_copy(
                    tT_hbm.at[:, pl.ds(cstart(n + 1), _VBLK)],
                    in_v.at[1 - buf, :, pl.ds(0, _VBLK)], isem,
                )

            @pl.when(n >= 2)
            def _():
                pltpu.make_async_copy(
                    tr_v.at[buf], out_hbm.at[pl.ds(0, _VBLK * _D)],
                    osem.at[buf],
                ).wait()

            bufv = jnp.full((_LANES,), buf, jnp.int32)

            @plsc.parallel_loop(0, _VBLK // 2, unroll=4)
            def _(i):
                for h in range(2):
                    col = jnp.full((_LANES,), 2 * i + h, jnp.int32)
                    for m in range(_D // _LANES):
                        v = plsc.load_gather(
                            in_v, [bufv, m * _LANES + lanes, col]
                        )
                        tr_v[buf, pl.ds(i * (2 * _D) + h * _D + m * _LANES,
                                        _LANES)] = v

            pltpu.async_copy(
                tr_v.at[buf],
                out_hbm.at[pl.ds((wid + _NW * n) * (_VBLK * _D), _VBLK * _D)],
                osem.at[buf],
            )
            return 0

        lax.fori_loop(0, nblk, blk, 0, unroll=False)
        for b in range(2):
            pltpu.make_async_copy(
                tr_v.at[b], out_hbm.at[pl.ds(0, _VBLK * _D)], osem.at[b]
            ).wait()

        @pl.when(wid == 0)
        def _():
            pltpu.sync_copy(tail_hbm, tail_v)
            pltpu.sync_copy(
                tail_v, out_hbm.at[pl.ds(_NBLK * _VBLK * _D, _VTAIL * _D)]
            )

    return call


def _make_sc_call():
    mesh = plsc.VectorSubcoreMesh(core_axis_name="c", subcore_axis_name="s")

    @functools.partial(
        pl.kernel,
        out_type=jax.ShapeDtypeStruct((_B, 3 * _D), jnp.float32),
        mesh=mesh,
        compiler_params=pltpu.CompilerParams(use_tc_tiling_on_sc=False),
        scratch_types=[
            pltpu.VMEM((_CH,), jnp.int32),
            pltpu.VMEM((4, _CH), jnp.int32),
            pltpu.VMEM((4, _CH), jnp.int32),
            pltpu.VMEM((_CH, _D), jnp.float32),
            pltpu.VMEM((4 * _CH, _D), jnp.float32),
            pltpu.VMEM((4 * _CH, _D), jnp.float32),
            pltpu.VMEM((_CH, _D), jnp.float32),
            pltpu.VMEM((_CH, _D), jnp.float32),
            pltpu.SemaphoreType.DMA,
        ],
    )
    def call(opc_hbm, op1_hbm, op2_hbm, table_hbm, out_hbm,
             idx0_v, idx1_v, idx2_v, rows0_v, rows1_v, rows2_v,
             acc1_v, acc2_v, sem):
        wid = lax.axis_index("s") * _NC + lax.axis_index("c")
        quarter = jnp.float32(0.25)

        for c in range(_NCHUNK):
            g = wid * _NCHUNK + c
            pltpu.sync_copy(opc_hbm.at[g], idx0_v)
            pltpu.sync_copy(op1_hbm.at[g], idx1_v)
            pltpu.sync_copy(op2_hbm.at[g], idx2_v)
            copies = [pltpu.async_copy(table_hbm.at[idx0_v], rows0_v, sem)]
            for q in range(4):
                copies.append(pltpu.async_copy(
                    table_hbm.at[idx1_v.at[q]],
                    rows1_v.at[pl.ds(q * _CH, _CH)], sem))
            for q in range(4):
                copies.append(pltpu.async_copy(
                    table_hbm.at[idx2_v.at[q]],
                    rows2_v.at[pl.ds(q * _CH, _CH)], sem))
            for cp in copies:
                cp.wait()
            pltpu.sync_copy(
                rows0_v, out_hbm.at[pl.ds(g * _CH, _CH), pl.ds(0, _D)]
            )

            def body(i, _):
                base = 4 * i
                for k in range(_D // _LANES):
                    s = pl.ds(k * _LANES, _LANES)
                    a1 = (
                        rows1_v[base, s] + rows1_v[base + 1, s]
                        + rows1_v[base + 2, s] + rows1_v[base + 3, s]
                    ) * quarter
                    acc1_v[i, s] = a1
                    a2 = (
                        rows2_v[base, s] + rows2_v[base + 1, s]
                        + rows2_v[base + 2, s] + rows2_v[base + 3, s]
                    ) * quarter
                    acc2_v[i, s] = a2
                return 0

            lax.fori_loop(0, _CH, body, 0, unroll=False)

            pltpu.sync_copy(
                acc1_v, out_hbm.at[pl.ds(g * _CH, _CH), pl.ds(_D, _D)]
            )
            pltpu.sync_copy(
                acc2_v, out_hbm.at[pl.ds(g * _CH, _CH), pl.ds(2 * _D, _D)]
            )

    return call


_transpose_call = _make_transpose_call()
_sc_call = _make_sc_call()


@jax.jit
def kernel(opcode_idx, op1_idx, op2_idx, table):
    tT = table.T
    tail = table[_NBLK * _VBLK:, :].astype(jnp.float32).reshape(-1)
    t_flat = _transpose_call(tT, tail)
    t_lin = t_flat.reshape(_VOCAB, _D)

    opc = opcode_idx.astype(jnp.int32).reshape(_G, _CH)
    op1 = op1_idx.astype(jnp.int32).reshape(_G, 4, _CH)
    op2 = op2_idx.astype(jnp.int32).reshape(_G, 4, _CH)
    return _sc_call(opc, op1, op2, t_lin)


# speedup vs baseline: 2.6075x; 2.6075x over previous
import functools

import jax
import jax.numpy as jnp
from jax import lax
from jax.experimental import pallas as pl
from jax.experimental.pallas import tpu as pltpu
from jax.experimental.pallas import tpu_sc as plsc

_VOCAB = 1000000
_D = 64
_B = 16384
_LANES = 16

_NC = 2
_NS = 16
_NW = _NC * _NS

_CH = 128
_NCHUNK = _B // (_NW * _CH)
_G = _NW * _NCHUNK

_VBLK = 384
_NBLK = _VOCAB // _VBLK
_VTAIL = _VOCAB - _NBLK * _VBLK
_BLK_W = -(-_NBLK // _NW)
_BLK_LO = _NBLK - (_BLK_W - 1) * _NW


def _make_transpose_call():
    mesh = plsc.VectorSubcoreMesh(core_axis_name="c", subcore_axis_name="s")

    @functools.partial(
        pl.kernel,
        out_type=jax.ShapeDtypeStruct((_VOCAB * _D,), jnp.float32),
        mesh=mesh,
        compiler_params=pltpu.CompilerParams(
            use_tc_tiling_on_sc=True, needs_layout_passes=False
        ),
        scratch_types=[
            pltpu.VMEM((2, _D, _VBLK), jnp.float32),
            pltpu.VMEM((2, _VBLK * _D), jnp.float32),
            pltpu.VMEM((_VTAIL * _D,), jnp.float32),
            pltpu.SemaphoreType.DMA,
            pltpu.SemaphoreType.DMA((2,)),
        ],
    )
    def call(tT_hbm, tail_hbm, out_hbm, in_v, tr_v, tail_v, isem, osem):
        wid = lax.axis_index("s") * _NC + lax.axis_index("c")
        nblk = jnp.where(wid < _BLK_LO, _BLK_W, _BLK_W - 1)
        lanes = lax.iota(jnp.int32, _LANES)

        def cstart(n):
            return pl.multiple_of((wid + _NW * n) * _VBLK, 128)

        pltpu.async_copy(
            tT_hbm.at[:, pl.ds(cstart(0), _VBLK)], in_v.at[0], isem
        )

        def blk(n, _):
            buf = lax.rem(n, 2)
            pltpu.make_async_copy(
                tT_hbm.at[:, pl.ds(0, _VBLK)], in_v.at[buf], isem
            ).wait()

            @pl.when(n + 1 < nblk)
            def _():
                pltpu.async_copy(
                    tT_hbm.at[:, pl.ds(cstart(n + 1), _VBLK)],
                    in_v.at[1 - buf], isem,
                )

            @pl.when(n >= 2)
            def _():
                pltpu.make_async_copy(
                    tr_v.at[buf], out_hbm.at[pl.ds(0, _VBLK * _D)],
                    osem.at[buf],
                ).wait()

            bufv = jnp.full((_LANES,), buf, jnp.int32)

            @plsc.parallel_loop(0, _VBLK // _LANES, unroll=2)
            def _(cb16):
                cb = cb16 * _LANES
                for s in range(_LANES):
                    rot = lax.rem(lanes + s, _LANES)
                    colv = cb + rot
                    for m in range(_D // _LANES):
                        v = plsc.load_gather(
                            in_v, [bufv, m * _LANES + lanes, colv]
                        )
                        pos = colv * _D + (m * _LANES + lanes)
                        plsc.store_scatter(tr_v, [bufv, pos], v)

            pltpu.async_copy(
                tr_v.at[buf],
                out_hbm.at[pl.ds((wid + _NW * n) * (_VBLK * _D), _VBLK * _D)],
                osem.at[buf],
            )
            return 0

        lax.fori_loop(0, nblk, blk, 0, unroll=False)
        for b in range(2):
            pltpu.make_async_copy(
                tr_v.at[b], out_hbm.at[pl.ds(0, _VBLK * _D)], osem.at[b]
            ).wait()

        @pl.when(wid == 0)
        def _():
            pltpu.sync_copy(tail_hbm, tail_v)
            pltpu.sync_copy(
                tail_v, out_hbm.at[pl.ds(_NBLK * _VBLK * _D, _VTAIL * _D)]
            )

    return call


def _make_sc_call():
    mesh = plsc.VectorSubcoreMesh(core_axis_name="c", subcore_axis_name="s")

    @functools.partial(
        pl.kernel,
        out_type=jax.ShapeDtypeStruct((_B, 3 * _D), jnp.float32),
        mesh=mesh,
        compiler_params=pltpu.CompilerParams(use_tc_tiling_on_sc=False),
        scratch_types=[
            pltpu.VMEM((_CH,), jnp.int32),
            pltpu.VMEM((4, _CH), jnp.int32),
            pltpu.VMEM((4, _CH), jnp.int32),
            pltpu.VMEM((_CH, _D), jnp.float32),
            pltpu.VMEM((4 * _CH, _D), jnp.float32),
            pltpu.VMEM((4 * _CH, _D), jnp.float32),
            pltpu.VMEM((_CH, _D), jnp.float32),
            pltpu.VMEM((_CH, _D), jnp.float32),
            pltpu.SemaphoreType.DMA,
        ],
    )
    def call(opc_hbm, op1_hbm, op2_hbm, table_hbm, out_hbm,
             idx0_v, idx1_v, idx2_v, rows0_v, rows1_v, rows2_v,
             acc1_v, acc2_v, sem):
        wid = lax.axis_index("s") * _NC + lax.axis_index("c")
        quarter = jnp.float32(0.25)

        for c in range(_NCHUNK):
            g = wid * _NCHUNK + c
            pltpu.sync_copy(opc_hbm.at[g], idx0_v)
            pltpu.sync_copy(op1_hbm.at[g], idx1_v)
            pltpu.sync_copy(op2_hbm.at[g], idx2_v)
            copies = [pltpu.async_copy(table_hbm.at[idx0_v], rows0_v, sem)]
            for q in range(4):
                copies.append(pltpu.async_copy(
                    table_hbm.at[idx1_v.at[q]],
                    rows1_v.at[pl.ds(q * _CH, _CH)], sem))
            for q in range(4):
                copies.append(pltpu.async_copy(
                    table_hbm.at[idx2_v.at[q]],
                    rows2_v.at[pl.ds(q * _CH, _CH)], sem))
            for cp in copies:
                cp.wait()
            pltpu.sync_copy(
                rows0_v, out_hbm.at[pl.ds(g * _CH, _CH), pl.ds(0, _D)]
            )

            def body(i, _):
                base = 4 * i
                for k in range(_D // _LANES):
                    s = pl.ds(k * _LANES, _LANES)
                    a1 = (
                        rows1_v[base, s] + rows1_v[base + 1, s]
                        + rows1_v[base + 2, s] + rows1_v[base + 3, s]
                    ) * quarter
                    acc1_v[i, s] = a1
                    a2 = (
                        rows2_v[base, s] + rows2_v[base + 1, s]
                        + rows2_v[base + 2, s] + rows2_v[base + 3, s]
                    ) * quarter
                    acc2_v[i, s] = a2
                return 0

            lax.fori_loop(0, _CH, body, 0, unroll=False)

            pltpu.sync_copy(
                acc1_v, out_hbm.at[pl.ds(g * _CH, _CH), pl.ds(_D, _D)]
            )
            pltpu.sync_copy(
                acc2_v, out_hbm.at[pl.ds(g * _CH, _CH), pl.ds(2 * _D, _D)]
            )

    return call


_transpose_call = _make_transpose_call()
_sc_call = _make_sc_call()


@jax.jit
def kernel(opcode_idx, op1_idx, op2_idx, table):
    tT = table.T
    tail = table[_NBLK * _VBLK:, :].astype(jnp.float32).reshape(-1)
    t_flat = _transpose_call(tT, tail)
    t_lin = t_flat.reshape(_VOCAB, _D)

    opc = opcode_idx.astype(jnp.int32).reshape(_G, _CH)
    op1 = op1_idx.astype(jnp.int32).reshape(_G, 4, _CH)
    op2 = op2_idx.astype(jnp.int32).reshape(_G, 4, _CH)
    return _sc_call(opc, op1, op2, t_lin)


# speedup vs baseline: 3.1177x; 1.1957x over previous
import functools

import jax
import jax.numpy as jnp
from jax import lax
from jax.experimental import pallas as pl
from jax.experimental.pallas import tpu as pltpu
from jax.experimental.pallas import tpu_sc as plsc

_VOCAB = 1000000
_D = 64
_B = 16384
_LANES = 16

_NC = 2
_NS = 16
_NW = _NC * _NS

_CH = 128
_NCHUNK = _B // (_NW * _CH)
_G = _NW * _NCHUNK

_VBLK = 384
_NBLK = _VOCAB // _VBLK
_VTAIL = _VOCAB - _NBLK * _VBLK
_BLK_W = -(-_NBLK // _NW)
_BLK_LO = _NBLK - (_BLK_W - 1) * _NW


def _make_transpose_call():
    mesh = plsc.VectorSubcoreMesh(core_axis_name="c", subcore_axis_name="s")

    @functools.partial(
        pl.kernel,
        out_type=jax.ShapeDtypeStruct((_VOCAB * _D,), jnp.float32),
        mesh=mesh,
        compiler_params=pltpu.CompilerParams(
            use_tc_tiling_on_sc=True, needs_layout_passes=False
        ),
        scratch_types=[
            pltpu.VMEM((2, _D, _VBLK), jnp.float32),
            pltpu.VMEM((2, _VBLK * _D), jnp.float32),
            pltpu.VMEM((_VTAIL * _D,), jnp.float32),
            pltpu.SemaphoreType.DMA,
            pltpu.SemaphoreType.DMA((2,)),
        ],
    )
    def call(tT_hbm, tail_hbm, out_hbm, in_v, tr_v, tail_v, isem, osem):
        wid = lax.axis_index("s") * _NC + lax.axis_index("c")
        nblk = jnp.where(wid < _BLK_LO, _BLK_W, _BLK_W - 1)
        lanes = lax.iota(jnp.int32, _LANES)

        def cstart(n):
            return pl.multiple_of((wid + _NW * n) * _VBLK, 128)

        pltpu.async_copy(
            tT_hbm.at[:, pl.ds(cstart(0), _VBLK)], in_v.at[0], isem
        )

        def blk(n, _):
            buf = lax.rem(n, 2)
            pltpu.make_async_copy(
                tT_hbm.at[:, pl.ds(0, _VBLK)], in_v.at[buf], isem
            ).wait()

            @pl.when(n + 1 < nblk)
            def _():
                pltpu.async_copy(
                    tT_hbm.at[:, pl.ds(cstart(n + 1), _VBLK)],
                    in_v.at[1 - buf], isem,
                )

            @pl.when(n >= 2)
            def _():
                pltpu.make_async_copy(
                    tr_v.at[buf], out_hbm.at[pl.ds(0, _VBLK * _D)],
                    osem.at[buf],
                ).wait()

            bufv = jnp.full((_LANES,), buf, jnp.int32)

            @plsc.parallel_loop(0, _VBLK // _LANES, unroll=4)
            def _(cb16):
                cb = cb16 * _LANES
                for s in range(_LANES):
                    rot = lax.rem(lanes + s, _LANES)
                    colv = cb + rot
                    for m in range(_D // _LANES):
                        v = plsc.load_gather(
                            in_v, [bufv, m * _LANES + lanes, colv]
                        )
                        pos = colv * _D + (m * _LANES + lanes)
                        plsc.store_scatter(tr_v, [bufv, pos], v)

            pltpu.async_copy(
                tr_v.at[buf],
                out_hbm.at[pl.ds((wid + _NW * n) * (_VBLK * _D), _VBLK * _D)],
                osem.at[buf],
            )
            return 0

        lax.fori_loop(0, nblk, blk, 0, unroll=False)
        for b in range(2):
            pltpu.make_async_copy(
                tr_v.at[b], out_hbm.at[pl.ds(0, _VBLK * _D)], osem.at[b]
            ).wait()

        @pl.when(wid == 0)
        def _():
            pltpu.sync_copy(tail_hbm, tail_v)
            pltpu.sync_copy(
                tail_v, out_hbm.at[pl.ds(_NBLK * _VBLK * _D, _VTAIL * _D)]
            )

    return call


def _make_sc_call():
    mesh = plsc.VectorSubcoreMesh(core_axis_name="c", subcore_axis_name="s")

    @functools.partial(
        pl.kernel,
        out_type=jax.ShapeDtypeStruct((_B, 3 * _D), jnp.float32),
        mesh=mesh,
        compiler_params=pltpu.CompilerParams(use_tc_tiling_on_sc=False),
        scratch_types=[
            pltpu.VMEM((_CH,), jnp.int32),
            pltpu.VMEM((4, _CH), jnp.int32),
            pltpu.VMEM((4, _CH), jnp.int32),
            pltpu.VMEM((_CH, _D), jnp.float32),
            pltpu.VMEM((4 * _CH, _D), jnp.float32),
            pltpu.VMEM((4 * _CH, _D), jnp.float32),
            pltpu.VMEM((_CH, _D), jnp.float32),
            pltpu.VMEM((_CH, _D), jnp.float32),
            pltpu.SemaphoreType.DMA,
        ],
    )
    def call(opc_hbm, op1_hbm, op2_hbm, table_hbm, out_hbm,
             idx0_v, idx1_v, idx2_v, rows0_v, rows1_v, rows2_v,
             acc1_v, acc2_v, sem):
        wid = lax.axis_index("s") * _NC + lax.axis_index("c")
        quarter = jnp.float32(0.25)

        for c in range(_NCHUNK):
            g = wid * _NCHUNK + c
            pltpu.sync_copy(opc_hbm.at[g], idx0_v)
            pltpu.sync_copy(op1_hbm.at[g], idx1_v)
            pltpu.sync_copy(op2_hbm.at[g], idx2_v)
            copies = [pltpu.async_copy(table_hbm.at[idx0_v], rows0_v, sem)]
            for q in range(4):
                copies.append(pltpu.async_copy(
                    table_hbm.at[idx1_v.at[q]],
                    rows1_v.at[pl.ds(q * _CH, _CH)], sem))
            for q in range(4):
                copies.append(pltpu.async_copy(
                    table_hbm.at[idx2_v.at[q]],
                    rows2_v.at[pl.ds(q * _CH, _CH)], sem))
            for cp in copies:
                cp.wait()
            pltpu.sync_copy(
                rows0_v, out_hbm.at[pl.ds(g * _CH, _CH), pl.ds(0, _D)]
            )

            @plsc.parallel_loop(0, _CH, unroll=2)
            def _(i):
                base = 4 * i
                for k in range(_D // _LANES):
                    s = pl.ds(k * _LANES, _LANES)
                    a1 = (
                        rows1_v[base, s] + rows1_v[base + 1, s]
                        + rows1_v[base + 2, s] + rows1_v[base + 3, s]
                    ) * quarter
                    acc1_v[i, s] = a1
                    a2 = (
                        rows2_v[base, s] + rows2_v[base + 1, s]
                        + rows2_v[base + 2, s] + rows2_v[base + 3, s]
                    ) * quarter
                    acc2_v[i, s] = a2

            pltpu.sync_copy(
                acc1_v, out_hbm.at[pl.ds(g * _CH, _CH), pl.ds(_D, _D)]
            )
            pltpu.sync_copy(
                acc2_v, out_hbm.at[pl.ds(g * _CH, _CH), pl.ds(2 * _D, _D)]
            )

    return call


_transpose_call = _make_transpose_call()
_sc_call = _make_sc_call()


@jax.jit
def kernel(opcode_idx, op1_idx, op2_idx, table):
    tT = table.T
    tail = table[_NBLK * _VBLK:, :].astype(jnp.float32).reshape(-1)
    t_flat = _transpose_call(tT, tail)
    t_lin = t_flat.reshape(_VOCAB, _D)

    opc = opcode_idx.astype(jnp.int32).reshape(_G, _CH)
    op1 = op1_idx.astype(jnp.int32).reshape(_G, 4, _CH)
    op2 = op2_idx.astype(jnp.int32).reshape(_G, 4, _CH)
    return _sc_call(opc, op1, op2, t_lin)
